# baseline (speedup 1.0000x reference)
import functools
import math

import jax
import jax.numpy as jnp
import numpy as np
from jax import lax
from jax.experimental import pallas as pl
from jax.experimental.pallas import tpu as pltpu
from jax.experimental.pallas import tpu_sc as plsc

_NUM_MASKS = 100
_MIN_CLUSTER = 3
_MAX_CLUSTER = 6
_SPAD = 16


@functools.cache
def _mask_params(B, H, W):
    with jax.ensure_compile_time_eval():
        kc, kp = jax.random.split(jax.random.key(42))
        cs = int(jax.random.randint(kc, (), _MIN_CLUSTER, _MAX_CLUSTER))
        mh, mw = min(H, cs), min(W, cs)
        S = math.ceil(_NUM_MASKS / (mh * mw))
        keys = jax.random.split(kp, B)
        idx = np.stack(
            [np.asarray(jax.random.permutation(keys[b], H * W))[:S] for b in range(B)]
        ).astype(np.int32)
    fh, fw = (mh - 1) // 2, (mw - 1) // 2
    ii = np.arange(H)[:, None]
    jj = np.arange(W)[None, :]
    mask = np.zeros((B, H, W), np.bool_)
    for b in range(B):
        for q in idx[b]:
            qi, qj = q // W, q % W
            mask[b] |= ((ii >= qi - fh) & (ii <= qi + mh - 1 - fh)
                        & (jj >= qj - fw) & (jj <= qj + mw - 1 - fw))
    mcol = mask.reshape(B, H * W, 1).astype(np.int8)
    idx_pad = np.zeros((B, _SPAD), np.int32)
    idx_pad[:, :S] = idx
    return idx_pad.reshape(-1), mcol, mh, mw, S


def _sc_mask_kernel(idx_hbm, mrow_hbm, row_v, idx_v, *, H, W, mh, mw, S, NC):
    HW = H * W
    b = lax.axis_index("s") * NC + lax.axis_index("c")
    pltpu.sync_copy(idx_hbm.at[pl.ds(b * _SPAD, _SPAD)], idx_v)
    q = idx_v[...]
    zero = jnp.zeros((16,), jnp.int32)
    for i in range(HW // 16):
        row_v[pl.ds(i * 16, 16)] = zero
    ones = jnp.ones((16,), jnp.int32)
    lane = lax.iota(jnp.int32, 16)
    fh = (mh - 1) // 2
    fw = (mw - 1) // 2
    qi = q // W
    qj = q % W
    for t in range(mh * mw):
        ii = qi - fh + (t // mw)
        jj = qj - fw + (t % mw)
        valid = ((lane < S) & (ii >= 0) & (ii < H)
                 & (jj >= 0) & (jj < W))
        p = ii * W + jj
        plsc.store_scatter(row_v, [p], ones, mask=valid)
    pltpu.sync_copy(row_v, mrow_hbm.at[pl.ds(b * HW, HW)])


def _masker_kernel(x_ref, mcol_ref, emb_ref, out_ref):
    sel = mcol_ref[...] != 0
    out_ref[...] = jnp.where(sel, emb_ref[...], x_ref[...])


def kernel(input, mask_embedding):
    B, C, H, W = input.shape
    idx_flat, mcol, mh, mw, S = _mask_params(B, H, W)
    HW = H * W
    info = plsc.get_sparse_core_info()
    NC = info.num_cores
    sc_call = functools.partial(
        pl.kernel,
        mesh=plsc.VectorSubcoreMesh(core_axis_name="c", subcore_axis_name="s"),
        out_type=jax.ShapeDtypeStruct((B * HW,), jnp.int32),
        scratch_types=[
            pltpu.VMEM((HW,), jnp.int32),
            pltpu.VMEM((_SPAD,), jnp.int32),
        ],
        compiler_params=pltpu.CompilerParams(needs_layout_passes=False),
    )(functools.partial(_sc_mask_kernel, H=H, W=W, mh=mh, mw=mw, S=S, NC=NC))
    mrow = sc_call(jnp.asarray(idx_flat))
    mask_out = mrow.reshape(B, H, W) > 0

    x = jnp.transpose(input, (0, 2, 3, 1)).reshape(B, HW, C)
    emb = mask_embedding.reshape(1, 1, C)
    BB = 4 if B % 4 == 0 else (2 if B % 2 == 0 else 1)
    out = pl.pallas_call(
        _masker_kernel,
        grid=(B // BB,),
        in_specs=[
            pl.BlockSpec((BB, HW, C), lambda b: (b, 0, 0)),
            pl.BlockSpec((BB, HW, 1), lambda b: (b, 0, 0)),
            pl.BlockSpec((1, 1, C), lambda b: (0, 0, 0)),
        ],
        out_specs=pl.BlockSpec((BB, HW, C), lambda b: (b, 0, 0)),
        out_shape=jax.ShapeDtypeStruct((B, HW, C), input.dtype),
    )(x, jnp.asarray(mcol), emb)
    out4 = jnp.transpose(out.reshape(B, H, W, C), (0, 3, 1, 2))
    return out4, mask_out
